# baseline (speedup 1.0000x reference)
import functools

import jax
import jax.numpy as jnp
from jax import lax
from jax.experimental import pallas as pl
from jax.experimental.pallas import tpu as pltpu
from jax.experimental.pallas import tpu_sc as plsc

N = 2048
D = 1024
H = 512
E = 8
BLK = 1024
NBLK = (N * 2) // BLK + E
P_PAD = NBLK * BLK
NW = 32
TPW = N // NW


def _router_body(x_ref, wg_ref, p0_ref, p1_ref, w0_ref, w1_ref, be_ref, nb_ref):
    xv = x_ref[...]
    wg = wg_ref[...]
    logits = jnp.dot(xv, wg, preferred_element_type=jnp.float32)
    lane = lax.broadcasted_iota(jnp.int32, (N, 16), 1)
    valid = lane < E
    lm = jnp.where(valid, logits, -1e30)
    m = jnp.max(lm, axis=1, keepdims=True)
    ex = jnp.where(valid, jnp.exp(lm - m), 0.0)
    p = ex / jnp.sum(ex, axis=1, keepdims=True)

    g0 = jnp.max(p, axis=1, keepdims=True)
    i0 = jnp.min(jnp.where(p >= g0, lane, 99), axis=1, keepdims=True)
    oh0 = lane == i0
    pm = jnp.where(oh0 | ~valid, -1.0, p)
    g1 = jnp.max(pm, axis=1, keepdims=True)
    i1 = jnp.min(jnp.where(pm >= g1, lane, 99), axis=1, keepdims=True)
    oh1 = lane == i1

    denom = g0 + g1 + 1e-6
    w0_ref[...] = g0 / denom
    w1_ref[...] = g1 / denom

    oh0f = oh0.astype(jnp.float32)
    oh1f = oh1.astype(jnp.float32)
    CH = 256
    r = lax.broadcasted_iota(jnp.int32, (CH, CH), 0)
    c = lax.broadcasted_iota(jnp.int32, (CH, CH), 1)
    tri = (c < r).astype(jnp.float32)
    parts0, parts1 = [], []
    carry0 = jnp.zeros((1, 16), jnp.float32)
    carry1 = jnp.zeros((1, 16), jnp.float32)
    for ch in range(N // CH):
        s0 = oh0f[ch * CH:(ch + 1) * CH]
        s1 = oh1f[ch * CH:(ch + 1) * CH]
        c0 = jnp.dot(tri, s0, preferred_element_type=jnp.float32) + carry0
        c1 = jnp.dot(tri, s1, preferred_element_type=jnp.float32) + carry1
        parts0.append(jnp.sum(c0 * s0, axis=1, keepdims=True))
        parts1.append(jnp.sum(c1 * s1, axis=1, keepdims=True))
        carry0 = carry0 + jnp.sum(s0, axis=0, keepdims=True)
        carry1 = carry1 + jnp.sum(s1, axis=0, keepdims=True)
    tot0 = carry0
    tot1 = carry1
    counts = tot0 + tot1
    pc = jnp.ceil(counts / BLK) * BLK
    le = lax.broadcasted_iota(jnp.int32, (16, 16), 0)
    ce = lax.broadcasted_iota(jnp.int32, (16, 16), 1)
    m16 = (le < ce).astype(jnp.float32)
    poff = jnp.dot(pc, m16, preferred_element_type=jnp.float32)

    rank0 = jnp.concatenate(parts0, axis=0)
    rank1 = (jnp.concatenate(parts1, axis=0)
             + jnp.sum(tot0 * oh1f, axis=1, keepdims=True))
    base0 = jnp.sum(poff * oh0f, axis=1, keepdims=True)
    base1 = jnp.sum(poff * oh1f, axis=1, keepdims=True)
    p0_ref[...] = (base0 + rank0).astype(jnp.int32)
    p1_ref[...] = (base1 + rank1).astype(jnp.int32)

    bstart = poff / BLK
    nbf = jnp.sum(pc, axis=1, keepdims=True) / BLK
    jb = lax.broadcasted_iota(jnp.int32, (128, 16), 0).astype(jnp.float32)
    jb = jnp.minimum(jb, nbf - 1.0)
    becnt = jnp.sum((bstart <= jb).astype(jnp.int32), axis=1, keepdims=True)
    be_ref[...] = jnp.clip(becnt - 1, 0, E - 1)
    nb_ref[...] = nbf.astype(jnp.int32)


def _router(x2d, wg):
    f32 = jnp.float32
    return pl.pallas_call(
        _router_body,
        out_shape=(
            jax.ShapeDtypeStruct((N, 1), jnp.int32),
            jax.ShapeDtypeStruct((N, 1), jnp.int32),
            jax.ShapeDtypeStruct((N, 1), f32),
            jax.ShapeDtypeStruct((N, 1), f32),
            jax.ShapeDtypeStruct((128, 1), jnp.int32),
            jax.ShapeDtypeStruct((1, 1), jnp.int32),
        ),
    )(x2d, wg)


@functools.cache
def _sc_mesh():
    return plsc.VectorSubcoreMesh(core_axis_name="c", subcore_axis_name="s")


@functools.cache
def _sc_dispatch_call():
    @functools.partial(
        pl.kernel,
        out_type=jax.ShapeDtypeStruct((P_PAD, D), jnp.float32),
        mesh=_sc_mesh(),
        scratch_types=[
            pltpu.VMEM((TPW,), jnp.int32),
            pltpu.VMEM((TPW,), jnp.int32),
            pltpu.VMEM((TPW, D), jnp.float32),
            pltpu.SemaphoreType.DMA,
        ],
    )
    def _sc_dispatch(x_hbm, p0_hbm, p1_hbm, xs_hbm, i0_v, i1_v, rows_v, sem):
        wid = lax.axis_index("s") * 2 + lax.axis_index("c")
        base = wid * TPW
        pltpu.sync_copy(p0_hbm.at[pl.ds(base, TPW)], i0_v)
        pltpu.sync_copy(p1_hbm.at[pl.ds(base, TPW)], i1_v)
        pltpu.sync_copy(x_hbm.at[pl.ds(base, TPW)], rows_v)
        c0 = pltpu.async_copy(rows_v, xs_hbm.at[i0_v], sem)
        c1 = pltpu.async_copy(rows_v, xs_hbm.at[i1_v], sem)
        c0.wait()
        c1.wait()

    return _sc_dispatch


@functools.cache
def _sc_combine_call():
    @functools.partial(
        pl.kernel,
        out_type=(
            jax.ShapeDtypeStruct((N, D), jnp.float32),
            jax.ShapeDtypeStruct((N, D), jnp.float32),
        ),
        mesh=_sc_mesh(),
        scratch_types=[
            pltpu.VMEM((TPW,), jnp.int32),
            pltpu.VMEM((TPW,), jnp.int32),
            pltpu.VMEM((TPW // 2, D), jnp.float32),
            pltpu.VMEM((TPW // 2, D), jnp.float32),
            pltpu.SemaphoreType.DMA,
        ],
    )
    def _sc_combine(outs_hbm, p0_hbm, p1_hbm, o0_hbm, o1_hbm,
                    i0_v, i1_v, buf_a, buf_b, sem):
        wid = lax.axis_index("s") * 2 + lax.axis_index("c")
        base = wid * TPW
        half = TPW // 2
        pltpu.sync_copy(p0_hbm.at[pl.ds(base, TPW)], i0_v)
        pltpu.sync_copy(p1_hbm.at[pl.ds(base, TPW)], i1_v)
        ga = pltpu.async_copy(outs_hbm.at[i0_v.at[pl.ds(0, half)]], buf_a, sem)
        gb = pltpu.async_copy(outs_hbm.at[i0_v.at[pl.ds(half, half)]], buf_b, sem)
        ga.wait()
        pltpu.sync_copy(buf_a, o0_hbm.at[pl.ds(base, half)])
        ga = pltpu.async_copy(outs_hbm.at[i1_v.at[pl.ds(0, half)]], buf_a, sem)
        gb.wait()
        pltpu.sync_copy(buf_b, o0_hbm.at[pl.ds(base + half, half)])
        gb = pltpu.async_copy(outs_hbm.at[i1_v.at[pl.ds(half, half)]], buf_b, sem)
        ga.wait()
        pltpu.sync_copy(buf_a, o1_hbm.at[pl.ds(base, half)])
        gb.wait()
        pltpu.sync_copy(buf_b, o1_hbm.at[pl.ds(base + half, half)])

    return _sc_combine


def _expert_body(be_ref, nb_ref, xs_ref, w1_ref, b1_ref, w2_ref, b2_ref, o_ref):
    j = pl.program_id(0)

    @pl.when(j < nb_ref[0])
    def _():
        xb = xs_ref[...]
        h = jnp.dot(xb, w1_ref[0], preferred_element_type=jnp.float32)
        h = h + b1_ref[0]
        h = 0.5 * h * (1.0 + lax.erf(h * 0.7071067811865476))
        o = jnp.dot(h, w2_ref[0], preferred_element_type=jnp.float32)
        o_ref[...] = o + b2_ref[0]


def _experts(be, nb, xs, w1, b1, w2, b2):
    grid_spec = pltpu.PrefetchScalarGridSpec(
        num_scalar_prefetch=2,
        grid=(NBLK,),
        in_specs=[
            pl.BlockSpec((BLK, D),
                         lambda j, be, nb: (jnp.minimum(j, nb[0] - 1), 0)),
            pl.BlockSpec((1, D, H), lambda j, be, nb: (be[j], 0, 0)),
            pl.BlockSpec((1, 1, H), lambda j, be, nb: (be[j], 0, 0)),
            pl.BlockSpec((1, H, D), lambda j, be, nb: (be[j], 0, 0)),
            pl.BlockSpec((1, 1, D), lambda j, be, nb: (be[j], 0, 0)),
        ],
        out_specs=pl.BlockSpec((BLK, D),
                               lambda j, be, nb: (jnp.minimum(j, nb[0] - 1), 0)),
    )
    return pl.pallas_call(
        _expert_body,
        grid_spec=grid_spec,
        out_shape=jax.ShapeDtypeStruct((P_PAD, D), jnp.float32),
    )(be, nb, xs, w1, b1, w2, b2)


def _mix_body(o0_ref, o1_ref, w0_ref, w1_ref, y_ref):
    y_ref[...] = w0_ref[...] * o0_ref[...] + w1_ref[...] * o1_ref[...]


def _mix(o0, o1, w0, w1):
    nb = 4
    rows = N // nb
    return pl.pallas_call(
        _mix_body,
        grid=(nb,),
        in_specs=[
            pl.BlockSpec((rows, D), lambda i: (i, 0)),
            pl.BlockSpec((rows, D), lambda i: (i, 0)),
            pl.BlockSpec((rows, 1), lambda i: (i, 0)),
            pl.BlockSpec((rows, 1), lambda i: (i, 0)),
        ],
        out_specs=pl.BlockSpec((rows, D), lambda i: (i, 0)),
        out_shape=jax.ShapeDtypeStruct((N, D), jnp.float32),
    )(o0, o1, w0, w1)


@jax.jit
def kernel(x, task_bh, Wg, W1, b1, W2, b2):
    bsz, length, d = x.shape
    x2d = x.reshape(N, D)
    p0, p1, w0, w1g, be, nbk = _router(x2d, Wg)
    p0f = p0.reshape(N)
    p1f = p1.reshape(N)
    xs = _sc_dispatch_call()(x2d, p0f, p1f)
    outs = _experts(
        be.reshape(128), nbk.reshape(1), xs,
        W1, b1.reshape(E, 1, H), W2, b2.reshape(E, 1, D),
    )
    o0, o1 = _sc_combine_call()(outs, p0f, p1f)
    y = _mix(o0, o1, w0, w1g)
    return (y.reshape(bsz, length, d), jnp.float32(0.0))


# speedup vs baseline: 1.0280x; 1.0280x over previous
import functools

import jax
import jax.numpy as jnp
from jax import lax
from jax.experimental import pallas as pl
from jax.experimental.pallas import tpu as pltpu
from jax.experimental.pallas import tpu_sc as plsc

N = 2048
D = 1024
H = 512
E = 8
BLK = 512
NBLK = (N * 2) // BLK + E
P_PAD = NBLK * BLK
NW = 32
TPW = N // NW


def _router_body(x_ref, wg_ref, p0_ref, p1_ref, w0_ref, w1_ref, be_ref, nb_ref):
    xv = x_ref[...]
    wg = wg_ref[...]
    logits = jnp.dot(xv, wg, preferred_element_type=jnp.float32)
    lane = lax.broadcasted_iota(jnp.int32, (N, 16), 1)
    valid = lane < E
    lm = jnp.where(valid, logits, -1e30)
    m = jnp.max(lm, axis=1, keepdims=True)
    ex = jnp.where(valid, jnp.exp(lm - m), 0.0)
    p = ex / jnp.sum(ex, axis=1, keepdims=True)

    g0 = jnp.max(p, axis=1, keepdims=True)
    i0 = jnp.min(jnp.where(p >= g0, lane, 99), axis=1, keepdims=True)
    oh0 = lane == i0
    pm = jnp.where(oh0 | ~valid, -1.0, p)
    g1 = jnp.max(pm, axis=1, keepdims=True)
    i1 = jnp.min(jnp.where(pm >= g1, lane, 99), axis=1, keepdims=True)
    oh1 = lane == i1

    denom = g0 + g1 + 1e-6
    w0_ref[...] = g0 / denom
    w1_ref[...] = g1 / denom

    oh0f = oh0.astype(jnp.float32)
    oh1f = oh1.astype(jnp.float32)
    CH = 256
    r = lax.broadcasted_iota(jnp.int32, (CH, CH), 0)
    c = lax.broadcasted_iota(jnp.int32, (CH, CH), 1)
    tri = (c < r).astype(jnp.float32)
    parts0, parts1 = [], []
    carry0 = jnp.zeros((1, 16), jnp.float32)
    carry1 = jnp.zeros((1, 16), jnp.float32)
    for ch in range(N // CH):
        s0 = oh0f[ch * CH:(ch + 1) * CH]
        s1 = oh1f[ch * CH:(ch + 1) * CH]
        c0 = jnp.dot(tri, s0, preferred_element_type=jnp.float32) + carry0
        c1 = jnp.dot(tri, s1, preferred_element_type=jnp.float32) + carry1
        parts0.append(jnp.sum(c0 * s0, axis=1, keepdims=True))
        parts1.append(jnp.sum(c1 * s1, axis=1, keepdims=True))
        carry0 = carry0 + jnp.sum(s0, axis=0, keepdims=True)
        carry1 = carry1 + jnp.sum(s1, axis=0, keepdims=True)
    tot0 = carry0
    tot1 = carry1
    counts = tot0 + tot1
    pc = jnp.ceil(counts / BLK) * BLK
    le = lax.broadcasted_iota(jnp.int32, (16, 16), 0)
    ce = lax.broadcasted_iota(jnp.int32, (16, 16), 1)
    m16 = (le < ce).astype(jnp.float32)
    poff = jnp.dot(pc, m16, preferred_element_type=jnp.float32)

    rank0 = jnp.concatenate(parts0, axis=0)
    rank1 = (jnp.concatenate(parts1, axis=0)
             + jnp.sum(tot0 * oh1f, axis=1, keepdims=True))
    base0 = jnp.sum(poff * oh0f, axis=1, keepdims=True)
    base1 = jnp.sum(poff * oh1f, axis=1, keepdims=True)
    p0_ref[...] = (base0 + rank0).astype(jnp.int32)
    p1_ref[...] = (base1 + rank1).astype(jnp.int32)

    bstart = poff / BLK
    nbf = jnp.sum(pc, axis=1, keepdims=True) / BLK
    jb = lax.broadcasted_iota(jnp.int32, (128, 16), 0).astype(jnp.float32)
    jb = jnp.minimum(jb, nbf - 1.0)
    becnt = jnp.sum((bstart <= jb).astype(jnp.int32), axis=1, keepdims=True)
    be_ref[...] = jnp.clip(becnt - 1, 0, E - 1)
    nb_ref[...] = nbf.astype(jnp.int32)


def _router(x2d, wg):
    f32 = jnp.float32
    return pl.pallas_call(
        _router_body,
        out_shape=(
            jax.ShapeDtypeStruct((N, 1), jnp.int32),
            jax.ShapeDtypeStruct((N, 1), jnp.int32),
            jax.ShapeDtypeStruct((N, 1), f32),
            jax.ShapeDtypeStruct((N, 1), f32),
            jax.ShapeDtypeStruct((128, 1), jnp.int32),
            jax.ShapeDtypeStruct((1, 1), jnp.int32),
        ),
    )(x2d, wg)


@functools.cache
def _sc_mesh():
    return plsc.VectorSubcoreMesh(core_axis_name="c", subcore_axis_name="s")


@functools.cache
def _sc_dispatch_call():
    @functools.partial(
        pl.kernel,
        out_type=jax.ShapeDtypeStruct((P_PAD, D), jnp.float32),
        mesh=_sc_mesh(),
        scratch_types=[
            pltpu.VMEM((TPW,), jnp.int32),
            pltpu.VMEM((TPW,), jnp.int32),
            pltpu.VMEM((TPW, D), jnp.float32),
            pltpu.SemaphoreType.DMA,
        ],
    )
    def _sc_dispatch(x_hbm, p0_hbm, p1_hbm, xs_hbm, i0_v, i1_v, rows_v, sem):
        wid = lax.axis_index("s") * 2 + lax.axis_index("c")
        base = wid * TPW
        pltpu.sync_copy(p0_hbm.at[pl.ds(base, TPW)], i0_v)
        pltpu.sync_copy(p1_hbm.at[pl.ds(base, TPW)], i1_v)
        pltpu.sync_copy(x_hbm.at[pl.ds(base, TPW)], rows_v)
        c0 = pltpu.async_copy(rows_v, xs_hbm.at[i0_v], sem)
        c1 = pltpu.async_copy(rows_v, xs_hbm.at[i1_v], sem)
        c0.wait()
        c1.wait()

    return _sc_dispatch


@functools.cache
def _sc_combine_call():
    @functools.partial(
        pl.kernel,
        out_type=(
            jax.ShapeDtypeStruct((N, D), jnp.float32),
            jax.ShapeDtypeStruct((N, D), jnp.float32),
        ),
        mesh=_sc_mesh(),
        scratch_types=[
            pltpu.VMEM((TPW,), jnp.int32),
            pltpu.VMEM((TPW,), jnp.int32),
            pltpu.VMEM((TPW // 2, D), jnp.float32),
            pltpu.VMEM((TPW // 2, D), jnp.float32),
            pltpu.SemaphoreType.DMA,
        ],
    )
    def _sc_combine(outs_hbm, p0_hbm, p1_hbm, o0_hbm, o1_hbm,
                    i0_v, i1_v, buf_a, buf_b, sem):
        wid = lax.axis_index("s") * 2 + lax.axis_index("c")
        base = wid * TPW
        half = TPW // 2
        pltpu.sync_copy(p0_hbm.at[pl.ds(base, TPW)], i0_v)
        pltpu.sync_copy(p1_hbm.at[pl.ds(base, TPW)], i1_v)
        ga = pltpu.async_copy(outs_hbm.at[i0_v.at[pl.ds(0, half)]], buf_a, sem)
        gb = pltpu.async_copy(outs_hbm.at[i0_v.at[pl.ds(half, half)]], buf_b, sem)
        ga.wait()
        pltpu.sync_copy(buf_a, o0_hbm.at[pl.ds(base, half)])
        ga = pltpu.async_copy(outs_hbm.at[i1_v.at[pl.ds(0, half)]], buf_a, sem)
        gb.wait()
        pltpu.sync_copy(buf_b, o0_hbm.at[pl.ds(base + half, half)])
        gb = pltpu.async_copy(outs_hbm.at[i1_v.at[pl.ds(half, half)]], buf_b, sem)
        ga.wait()
        pltpu.sync_copy(buf_a, o1_hbm.at[pl.ds(base, half)])
        gb.wait()
        pltpu.sync_copy(buf_b, o1_hbm.at[pl.ds(base + half, half)])

    return _sc_combine


def _expert_body(be_ref, nb_ref, xs_ref, w1_ref, b1_ref, w2_ref, b2_ref, o_ref):
    j = pl.program_id(0)

    @pl.when(j < nb_ref[0])
    def _():
        xb = xs_ref[...]
        h = jnp.dot(xb, w1_ref[0], preferred_element_type=jnp.float32)
        h = h + b1_ref[0]
        h = 0.5 * h * (1.0 + lax.erf(h * 0.7071067811865476))
        o = jnp.dot(h, w2_ref[0], preferred_element_type=jnp.float32)
        o_ref[...] = o + b2_ref[0]


def _experts(be, nb, xs, w1, b1, w2, b2):
    grid_spec = pltpu.PrefetchScalarGridSpec(
        num_scalar_prefetch=2,
        grid=(NBLK,),
        in_specs=[
            pl.BlockSpec((BLK, D),
                         lambda j, be, nb: (jnp.minimum(j, nb[0] - 1), 0)),
            pl.BlockSpec((1, D, H), lambda j, be, nb: (be[j], 0, 0)),
            pl.BlockSpec((1, 1, H), lambda j, be, nb: (be[j], 0, 0)),
            pl.BlockSpec((1, H, D), lambda j, be, nb: (be[j], 0, 0)),
            pl.BlockSpec((1, 1, D), lambda j, be, nb: (be[j], 0, 0)),
        ],
        out_specs=pl.BlockSpec((BLK, D),
                               lambda j, be, nb: (jnp.minimum(j, nb[0] - 1), 0)),
    )
    return pl.pallas_call(
        _expert_body,
        grid_spec=grid_spec,
        out_shape=jax.ShapeDtypeStruct((P_PAD, D), jnp.float32),
    )(be, nb, xs, w1, b1, w2, b2)


def _mix_body(o0_ref, o1_ref, w0_ref, w1_ref, y_ref):
    y_ref[...] = w0_ref[...] * o0_ref[...] + w1_ref[...] * o1_ref[...]


def _mix(o0, o1, w0, w1):
    nb = 4
    rows = N // nb
    return pl.pallas_call(
        _mix_body,
        grid=(nb,),
        in_specs=[
            pl.BlockSpec((rows, D), lambda i: (i, 0)),
            pl.BlockSpec((rows, D), lambda i: (i, 0)),
            pl.BlockSpec((rows, 1), lambda i: (i, 0)),
            pl.BlockSpec((rows, 1), lambda i: (i, 0)),
        ],
        out_specs=pl.BlockSpec((rows, D), lambda i: (i, 0)),
        out_shape=jax.ShapeDtypeStruct((N, D), jnp.float32),
    )(o0, o1, w0, w1)


@jax.jit
def kernel(x, task_bh, Wg, W1, b1, W2, b2):
    bsz, length, d = x.shape
    x2d = x.reshape(N, D)
    p0, p1, w0, w1g, be, nbk = _router(x2d, Wg)
    p0f = p0.reshape(N)
    p1f = p1.reshape(N)
    xs = _sc_dispatch_call()(x2d, p0f, p1f)
    outs = _experts(
        be.reshape(128), nbk.reshape(1), xs,
        W1, b1.reshape(E, 1, H), W2, b2.reshape(E, 1, D),
    )
    o0, o1 = _sc_combine_call()(outs, p0f, p1f)
    y = _mix(o0, o1, w0, w1g)
    return (y.reshape(bsz, length, d), jnp.float32(0.0))
